# Initial kernel scaffold; baseline (speedup 1.0000x reference)
import jax
import jax.numpy as jnp
from jax.experimental import pallas as pl


def kernel(outputs, classes, emb, ids, w):
    raise NotImplementedError("write your pallas kernel here")


# speedup vs baseline: 1625.3248x; 1625.3248x over previous
import math

import jax
import jax.numpy as jnp
from jax.experimental import pallas as pl

B = 1024
D = 128
ALPHA = 0.1
BIG_M = 0.5
SMALL_M = 0.5
_COSM = math.cos(SMALL_M)
_SINM = math.sin(SMALL_M)


def _loss_kernel(cls_c_ref, cls_r_ref, ids_c_ref, ids_r_ref, emb_ref, w_ref,
                 out_ref):
    emb = emb_ref[...]
    w = w_ref[...]
    cls_c = cls_c_ref[...]

    nw2 = jnp.sum(w * w)
    ne2 = jnp.sum(emb * emb, axis=1, keepdims=True)
    embw = jnp.sum(emb * w, axis=1, keepdims=True)
    scale2 = nw2 * ne2
    sin_part = jnp.sqrt(jnp.maximum(scale2 - embw * embw, 0.0))
    outs = _COSM * embw + _SINM * (1.0 - 2.0 * cls_c) * sin_part
    bce = jnp.mean(jnp.maximum(outs, 0.0) - outs * cls_c
                   + jnp.log1p(jnp.exp(-jnp.abs(outs))))

    inv_norm = jax.lax.rsqrt(jnp.maximum(ne2, 1e-24))
    embn = emb * inv_norm
    gram = jax.lax.dot_general(embn, embn,
                               (((1,), (1,)), ((), ())),
                               preferred_element_type=jnp.float32)
    d = jnp.sqrt(jnp.maximum(2.0 - 2.0 * gram, 0.0))

    row = jax.lax.broadcasted_iota(jnp.int32, (B, B), 0)
    col = jax.lax.broadcasted_iota(jnp.int32, (B, B), 1)
    triu = col > row
    ids_eq = ids_c_ref[...] == ids_r_ref[...]
    cls_ne = cls_c != cls_r_ref[...]

    m1 = jnp.where(triu & ids_eq & cls_ne, 1.0, 0.0)
    m2 = jnp.where(triu & (~ids_eq) & (~cls_ne), 1.0, 0.0)
    s1 = jnp.sum(m1)
    s2 = jnp.sum(m2)
    sum1 = jnp.sum(m1 * d)
    sum2 = jnp.sum(m2 * jnp.maximum(0.0, BIG_M - d))
    l = jnp.where(s1 > 0, sum1 / jnp.maximum(s1, 1.0), 0.0)
    l = l + jnp.where(s2 > 0, sum2 / jnp.maximum(s2, 1.0), 0.0)

    out_ref[...] = jnp.broadcast_to(bce + ALPHA * l, (1, 1))


def kernel(outputs, classes, emb, ids, w):
    del outputs
    cls_c = classes.reshape(B, 1).astype(jnp.float32)
    cls_r = classes.reshape(1, B).astype(jnp.float32)
    ids_i = ids.astype(jnp.int32)
    ids_c = ids_i.reshape(B, 1)
    ids_r = ids_i.reshape(1, B)
    out = pl.pallas_call(
        _loss_kernel,
        out_shape=jax.ShapeDtypeStruct((1, 1), jnp.float32),
    )(cls_c, cls_r, ids_c, ids_r, emb, w)
    return out.reshape(())
